# Initial kernel scaffold; baseline (speedup 1.0000x reference)
import jax
import jax.numpy as jnp
from jax.experimental import pallas as pl


def kernel(x, edge_attr, edge_index, W_m1, b_m1, W_m2, b_m2, W_u1, b_u1, W_u2, b_u2):
    raise NotImplementedError("write your pallas kernel here")


# speedup vs baseline: 4.6157x; 4.6157x over previous
import functools

import jax
import jax.numpy as jnp
from jax import lax
from jax.experimental import pallas as pl
from jax.experimental.pallas import tpu as pltpu
from jax.experimental.pallas import tpu_sc as plsc

NC = 2
NS = 16
NW = NC * NS
CH = 128
ND = 10


def _sc_mesh():
    return plsc.VectorSubcoreMesh(core_axis_name="c", subcore_axis_name="s",
                                  num_cores=NC, num_subcores=NS)


def _gather_diff_count(x, i0, i1, zeros_nd):
    e2 = i0.shape[0]
    n, d = x.shape
    nchunk = e2 // CH
    rs = n // ND

    @functools.partial(
        pl.kernel,
        out_type=(
            jax.ShapeDtypeStruct((e2, d), x.dtype),
            jax.ShapeDtypeStruct((NC, n, d), jnp.float32),
        ),
        mesh=_sc_mesh(),
        scratch_types=[
            pltpu.VMEM((CH,), jnp.int32),
            pltpu.VMEM((CH,), jnp.int32),
            pltpu.VMEM((CH, d), x.dtype),
            pltpu.VMEM((CH, d), x.dtype),
            pltpu.VMEM((CH, d), jnp.float32),
            pltpu.VMEM_SHARED((n, d), jnp.float32),
            pltpu.SemaphoreType.DMA,
            pltpu.SemaphoreType.DMA,
        ],
    )
    def kern(x_hbm, i0_hbm, i1_hbm, z_hbm, out_hbm, cnt_hbm,
             i0_v, i1_v, r0_v, r1_v, ones_v, cnt_sh, s0, s1):
        cid = lax.axis_index("c")
        sid = lax.axis_index("s")
        wid = sid * NC + cid

        @pl.when(sid == 0)
        def _():
            pltpu.sync_copy(z_hbm, cnt_sh)

        one = jnp.full((16,), 1.0, jnp.float32)

        @pl.loop(0, CH)
        def _(r):
            @pl.loop(0, d, step=16)
            def _(k):
                ones_v[r, pl.ds(k, 16)] = one

        plsc.subcore_barrier()

        @pl.loop(wid, nchunk, step=NW)
        def _(c):
            base = c * CH
            pltpu.sync_copy(i0_hbm.at[pl.ds(base, CH)], i0_v)
            pltpu.sync_copy(i1_hbm.at[pl.ds(base, CH)], i1_v)
            cp1 = pltpu.async_copy(x_hbm.at[i1_v], r1_v, s1)
            cp0 = pltpu.async_copy(x_hbm.at[i0_v], r0_v, s0)
            pltpu.sync_copy(ones_v, cnt_sh.at[i1_v], add=True)
            cp1.wait()
            cp0.wait()

            @pl.loop(0, CH)
            def _(r):
                @pl.loop(0, d, step=16)
                def _(k):
                    sl = (r, pl.ds(k, 16))
                    r1_v[sl] = r1_v[sl] - r0_v[sl]

            pltpu.sync_copy(r1_v, out_hbm.at[pl.ds(base, CH)])

        plsc.subcore_barrier()

        @pl.when(sid < ND)
        def _():
            r0 = sid * rs
            pltpu.sync_copy(cnt_sh.at[pl.ds(r0, rs)], cnt_hbm.at[cid, pl.ds(r0, rs)])

    return kern(x, i0, i1, zeros_nd)


def _message_mlp(diff, ea, w1d, w1e, b1, w2, b2, n_pos_blocks, bm):
    e2, d = diff.shape
    grid = (e2 // bm,)
    de = ea.shape[1]
    h = w1d.shape[1]
    dm = w2.shape[1]

    def body(d_ref, ea_ref, w1d_ref, w1e_ref, b1_ref, w2_ref, b2_ref, o_ref):
        i = pl.program_id(0)
        sgn = jnp.where(i < n_pos_blocks, 1.0, -1.0).astype(jnp.float32)
        g = jnp.dot(d_ref[...], w1d_ref[...], preferred_element_type=jnp.float32)
        q = jnp.dot(ea_ref[...], w1e_ref[...], preferred_element_type=jnp.float32)
        g = g + sgn * q + b1_ref[...]
        hh = jnp.maximum(g, 0.0)
        m = jnp.dot(hh, w2_ref[...], preferred_element_type=jnp.float32) + b2_ref[...]
        o_ref[...] = jnp.maximum(m, 0.0)

    return pl.pallas_call(
        body,
        grid=grid,
        in_specs=[
            pl.BlockSpec((bm, d), lambda i: (i, 0)),
            pl.BlockSpec((bm, de),
                         lambda i: (jnp.where(i < n_pos_blocks, i, i - n_pos_blocks), 0)),
            pl.BlockSpec((d, h), lambda i: (0, 0)),
            pl.BlockSpec((de, h), lambda i: (0, 0)),
            pl.BlockSpec((1, h), lambda i: (0, 0)),
            pl.BlockSpec((h, dm), lambda i: (0, 0)),
            pl.BlockSpec((1, dm), lambda i: (0, 0)),
        ],
        out_specs=pl.BlockSpec((bm, dm), lambda i: (i, 0)),
        out_shape=jax.ShapeDtypeStruct((e2, dm), jnp.float32),
    )(diff, ea, w1d, w1e, b1, w2, b2)


def _scatter_agg(m, i1, zacc):
    e2, dm = m.shape
    n = zacc.shape[0]
    nchunk = e2 // CH
    rs = n // ND

    @functools.partial(
        pl.kernel,
        out_type=jax.ShapeDtypeStruct((NC, n, dm), jnp.float32),
        mesh=_sc_mesh(),
        scratch_types=[
            pltpu.VMEM((CH,), jnp.int32),
            pltpu.VMEM((CH, dm), jnp.float32),
            pltpu.VMEM_SHARED((n, dm), jnp.float32),
            pltpu.SemaphoreType.DMA,
        ],
    )
    def kern(m_hbm, i1_hbm, zacc_hbm, agg_hbm, idx_v, rows_v, acc_sh, sem):
        cid = lax.axis_index("c")
        sid = lax.axis_index("s")
        wid = sid * NC + cid

        @pl.when(sid == 0)
        def _():
            pltpu.sync_copy(zacc_hbm, acc_sh)

        plsc.subcore_barrier()

        @pl.loop(wid, nchunk, step=NW)
        def _(c):
            base = c * CH
            pltpu.sync_copy(i1_hbm.at[pl.ds(base, CH)], idx_v)
            pltpu.sync_copy(m_hbm.at[pl.ds(base, CH)], rows_v)
            pltpu.sync_copy(rows_v, acc_sh.at[idx_v], add=True)

        plsc.subcore_barrier()

        @pl.when(sid < ND)
        def _():
            r0 = sid * rs
            pltpu.sync_copy(acc_sh.at[pl.ds(r0, rs)], agg_hbm.at[cid, pl.ds(r0, rs)])

    return kern(m, i1, zacc)


def _update_mlp(x, aggp, cntp, wu1a, wu1b, b1, wu2, b2, bn):
    n, d = x.shape
    dm = aggp.shape[2]
    h = wu1a.shape[1]
    dout = wu2.shape[1]
    grid = (n // bn,)

    def body(x_ref, a_ref, c_ref, w1a_ref, w1b_ref, b1_ref, w2_ref, b2_ref, o_ref):
        agg = a_ref[0] + a_ref[1]
        cnt = c_ref[0][:, 0:1] + c_ref[1][:, 0:1]
        aggm = agg / jnp.maximum(cnt, 1.0)
        g = (jnp.dot(x_ref[...], w1a_ref[...], preferred_element_type=jnp.float32)
             + jnp.dot(aggm, w1b_ref[...], preferred_element_type=jnp.float32)
             + b1_ref[...])
        u = jnp.maximum(g, 0.0)
        o_ref[...] = jnp.dot(u, w2_ref[...], preferred_element_type=jnp.float32) + b2_ref[...]

    return pl.pallas_call(
        body,
        grid=grid,
        in_specs=[
            pl.BlockSpec((bn, d), lambda i: (i, 0)),
            pl.BlockSpec((NC, bn, dm), lambda i: (0, i, 0)),
            pl.BlockSpec((NC, bn, dm), lambda i: (0, i, 0)),
            pl.BlockSpec((d, h), lambda i: (0, 0)),
            pl.BlockSpec((dm, h), lambda i: (0, 0)),
            pl.BlockSpec((1, h), lambda i: (0, 0)),
            pl.BlockSpec((h, dout), lambda i: (0, 0)),
            pl.BlockSpec((1, dout), lambda i: (0, 0)),
        ],
        out_specs=pl.BlockSpec((bn, dout), lambda i: (i, 0)),
        out_shape=jax.ShapeDtypeStruct((n, dout), jnp.float32),
    )(x, aggp, cntp, wu1a, wu1b, b1, wu2, b2)


def kernel(x, edge_attr, edge_index, W_m1, b_m1, W_m2, b_m2, W_u1, b_u1, W_u2, b_u2):
    n, d = x.shape
    e = edge_attr.shape[0]

    i0 = jnp.concatenate([edge_index[0], edge_index[1]], axis=0)
    i1 = jnp.concatenate([edge_index[1], edge_index[0]], axis=0)

    zeros_nd = jnp.zeros((n, d), jnp.float32)

    diff, cntp = _gather_diff_count(x, i0, i1, zeros_nd)

    bm = 2000
    m = _message_mlp(diff, edge_attr,
                     W_m1[:d], W_m1[d:], b_m1.reshape(1, -1),
                     W_m2, b_m2.reshape(1, -1),
                     n_pos_blocks=e // bm, bm=bm)

    aggp = _scatter_agg(m, i1, zeros_nd)

    out = _update_mlp(x, aggp, cntp,
                      W_u1[:d], W_u1[d:], b_u1.reshape(1, -1),
                      W_u2, b_u2.reshape(1, -1), bn=1000)
    return (out, None, None)


# speedup vs baseline: 4.6952x; 1.0172x over previous
import functools

import jax
import jax.numpy as jnp
from jax import lax
from jax.experimental import pallas as pl
from jax.experimental.pallas import tpu as pltpu
from jax.experimental.pallas import tpu_sc as plsc

NC = 2
NS = 16
NW = NC * NS
CH = 128
ND = 10


def _sc_mesh():
    return plsc.VectorSubcoreMesh(core_axis_name="c", subcore_axis_name="s",
                                  num_cores=NC, num_subcores=NS)


def _gather_diff_count(x, i0, i1, zeros_nd):
    e2 = i0.shape[0]
    n, d = x.shape
    nchunk = e2 // CH
    rs = n // ND

    @functools.partial(
        pl.kernel,
        out_type=(
            jax.ShapeDtypeStruct((e2, d), x.dtype),
            jax.ShapeDtypeStruct((NC, n, d), jnp.float32),
        ),
        mesh=_sc_mesh(),
        scratch_types=[
            pltpu.VMEM((CH,), jnp.int32),
            pltpu.VMEM((CH,), jnp.int32),
            pltpu.VMEM((CH, d), x.dtype),
            pltpu.VMEM((CH, d), x.dtype),
            pltpu.VMEM((CH, d), jnp.float32),
            pltpu.VMEM_SHARED((n, d), jnp.float32),
            pltpu.SemaphoreType.DMA,
            pltpu.SemaphoreType.DMA,
        ],
    )
    def kern(x_hbm, i0_hbm, i1_hbm, z_hbm, out_hbm, cnt_hbm,
             i0_v, i1_v, r0_v, r1_v, ones_v, cnt_sh, s0, s1):
        cid = lax.axis_index("c")
        sid = lax.axis_index("s")
        wid = sid * NC + cid

        @pl.when(sid == 0)
        def _():
            pltpu.sync_copy(z_hbm, cnt_sh)

        one = jnp.full((16,), 1.0, jnp.float32)

        @pl.loop(0, CH)
        def _(r):
            @pl.loop(0, d, step=16)
            def _(k):
                ones_v[r, pl.ds(k, 16)] = one

        plsc.subcore_barrier()

        @pl.loop(wid, nchunk, step=NW)
        def _(c):
            base = c * CH
            pltpu.sync_copy(i0_hbm.at[pl.ds(base, CH)], i0_v)
            pltpu.sync_copy(i1_hbm.at[pl.ds(base, CH)], i1_v)
            cp1 = pltpu.async_copy(x_hbm.at[i1_v], r1_v, s1)
            cp0 = pltpu.async_copy(x_hbm.at[i0_v], r0_v, s0)
            pltpu.sync_copy(ones_v, cnt_sh.at[i1_v], add=True)
            cp1.wait()
            cp0.wait()

            if x.dtype == jnp.bfloat16:
                @pl.loop(0, CH, step=2)
                def _(r):
                    @pl.loop(0, d, step=16)
                    def _(k):
                        sl = (pl.ds(pl.multiple_of(r, 2), 2), pl.ds(k, 16))
                        r1_v[sl] = r1_v[sl] - r0_v[sl]
            else:
                @pl.loop(0, CH)
                def _(r):
                    @pl.loop(0, d, step=16)
                    def _(k):
                        sl = (r, pl.ds(k, 16))
                        r1_v[sl] = r1_v[sl] - r0_v[sl]

            pltpu.sync_copy(r1_v, out_hbm.at[pl.ds(base, CH)])

        plsc.subcore_barrier()

        @pl.when(sid < ND)
        def _():
            r0 = sid * rs
            pltpu.sync_copy(cnt_sh.at[pl.ds(r0, rs)], cnt_hbm.at[cid, pl.ds(r0, rs)])

    return kern(x, i0, i1, zeros_nd)


def _message_mlp(diff, ea, w1d, w1e, b1, w2, b2, n_pos_blocks, bm):
    e2, d = diff.shape
    grid = (e2 // bm,)
    de = ea.shape[1]
    h = w1d.shape[1]
    dm = w2.shape[1]

    def body(d_ref, ea_ref, w1d_ref, w1e_ref, b1_ref, w2_ref, b2_ref, o_ref):
        i = pl.program_id(0)
        sgn = jnp.where(i < n_pos_blocks, 1.0, -1.0).astype(jnp.float32)
        g = jnp.dot(d_ref[...].astype(w1d_ref.dtype), w1d_ref[...],
                    preferred_element_type=jnp.float32)
        q = jnp.dot(ea_ref[...], w1e_ref[...], preferred_element_type=jnp.float32)
        g = g + sgn * q + b1_ref[...]
        hh = jnp.maximum(g, 0.0).astype(w2_ref.dtype)
        m = jnp.dot(hh, w2_ref[...], preferred_element_type=jnp.float32) + b2_ref[...]
        o_ref[...] = jnp.maximum(m, 0.0)

    return pl.pallas_call(
        body,
        grid=grid,
        in_specs=[
            pl.BlockSpec((bm, d), lambda i: (i, 0)),
            pl.BlockSpec((bm, de),
                         lambda i: (jnp.where(i < n_pos_blocks, i, i - n_pos_blocks), 0)),
            pl.BlockSpec((d, h), lambda i: (0, 0)),
            pl.BlockSpec((de, h), lambda i: (0, 0)),
            pl.BlockSpec((1, h), lambda i: (0, 0)),
            pl.BlockSpec((h, dm), lambda i: (0, 0)),
            pl.BlockSpec((1, dm), lambda i: (0, 0)),
        ],
        out_specs=pl.BlockSpec((bm, dm), lambda i: (i, 0)),
        out_shape=jax.ShapeDtypeStruct((e2, dm), jnp.float32),
    )(diff, ea, w1d, w1e, b1, w2, b2)


def _scatter_agg(m, i1, zacc):
    e2, dm = m.shape
    n = zacc.shape[0]
    nchunk = e2 // CH
    rs = n // ND

    @functools.partial(
        pl.kernel,
        out_type=jax.ShapeDtypeStruct((NC, n, dm), jnp.float32),
        mesh=_sc_mesh(),
        scratch_types=[
            pltpu.VMEM((CH,), jnp.int32),
            pltpu.VMEM((CH, dm), jnp.float32),
            pltpu.VMEM_SHARED((n, dm), jnp.float32),
            pltpu.SemaphoreType.DMA,
        ],
    )
    def kern(m_hbm, i1_hbm, zacc_hbm, agg_hbm, idx_v, rows_v, acc_sh, sem):
        cid = lax.axis_index("c")
        sid = lax.axis_index("s")
        wid = sid * NC + cid

        @pl.when(sid == 0)
        def _():
            pltpu.sync_copy(zacc_hbm, acc_sh)

        plsc.subcore_barrier()

        @pl.loop(wid, nchunk, step=NW)
        def _(c):
            base = c * CH
            pltpu.sync_copy(i1_hbm.at[pl.ds(base, CH)], idx_v)
            pltpu.sync_copy(m_hbm.at[pl.ds(base, CH)], rows_v)
            pltpu.sync_copy(rows_v, acc_sh.at[idx_v], add=True)

        plsc.subcore_barrier()

        @pl.when(sid < ND)
        def _():
            r0 = sid * rs
            pltpu.sync_copy(acc_sh.at[pl.ds(r0, rs)], agg_hbm.at[cid, pl.ds(r0, rs)])

    return kern(m, i1, zacc)


def _update_mlp(x, aggp, cntp, wu1a, wu1b, b1, wu2, b2, bn):
    n, d = x.shape
    dm = aggp.shape[2]
    h = wu1a.shape[1]
    dout = wu2.shape[1]
    grid = (n // bn,)

    def body(x_ref, a_ref, c_ref, w1a_ref, w1b_ref, b1_ref, w2_ref, b2_ref, o_ref):
        agg = a_ref[0] + a_ref[1]
        cnt = c_ref[0][:, 0:1] + c_ref[1][:, 0:1]
        aggm = (agg / jnp.maximum(cnt, 1.0)).astype(w1b_ref.dtype)
        g = (jnp.dot(x_ref[...], w1a_ref[...], preferred_element_type=jnp.float32)
             + jnp.dot(aggm, w1b_ref[...], preferred_element_type=jnp.float32)
             + b1_ref[...])
        u = jnp.maximum(g, 0.0).astype(w2_ref.dtype)
        o_ref[...] = jnp.dot(u, w2_ref[...], preferred_element_type=jnp.float32) + b2_ref[...]

    return pl.pallas_call(
        body,
        grid=grid,
        in_specs=[
            pl.BlockSpec((bn, d), lambda i: (i, 0)),
            pl.BlockSpec((NC, bn, dm), lambda i: (0, i, 0)),
            pl.BlockSpec((NC, bn, dm), lambda i: (0, i, 0)),
            pl.BlockSpec((d, h), lambda i: (0, 0)),
            pl.BlockSpec((dm, h), lambda i: (0, 0)),
            pl.BlockSpec((1, h), lambda i: (0, 0)),
            pl.BlockSpec((h, dout), lambda i: (0, 0)),
            pl.BlockSpec((1, dout), lambda i: (0, 0)),
        ],
        out_specs=pl.BlockSpec((bn, dout), lambda i: (i, 0)),
        out_shape=jax.ShapeDtypeStruct((n, dout), jnp.float32),
    )(x, aggp, cntp, wu1a, wu1b, b1, wu2, b2)


def kernel(x, edge_attr, edge_index, W_m1, b_m1, W_m2, b_m2, W_u1, b_u1, W_u2, b_u2):
    n, d = x.shape
    e = edge_attr.shape[0]

    i0 = jnp.concatenate([edge_index[0], edge_index[1]], axis=0)
    i1 = jnp.concatenate([edge_index[1], edge_index[0]], axis=0)

    zeros_nd = jnp.zeros((n, d), jnp.float32)
    bf = jnp.bfloat16
    xb = x.astype(bf)

    diff, cntp = _gather_diff_count(x, i0, i1, zeros_nd)

    bm = 2000
    m = _message_mlp(diff, edge_attr.astype(bf),
                     W_m1[:d].astype(bf), W_m1[d:].astype(bf), b_m1.reshape(1, -1),
                     W_m2.astype(bf), b_m2.reshape(1, -1),
                     n_pos_blocks=e // bm, bm=bm)

    aggp = _scatter_agg(m, i1, zeros_nd)

    out = _update_mlp(xb, aggp, cntp,
                      W_u1[:d].astype(bf), W_u1[d:].astype(bf), b_u1.reshape(1, -1),
                      W_u2.astype(bf), b_u2.reshape(1, -1), bn=1000)
    return (out, None, None)


# speedup vs baseline: 9.0077x; 1.9185x over previous
import functools

import jax
import jax.numpy as jnp
from jax import lax
from jax.experimental import pallas as pl
from jax.experimental.pallas import tpu as pltpu
from jax.experimental.pallas import tpu_sc as plsc

NC = 2
NS = 16
NW = NC * NS
CH = 128
ND = 10
DR = 1000
CZ = 200
PK = 128
CZ2 = 40


def _sc_mesh():
    return plsc.VectorSubcoreMesh(core_axis_name="c", subcore_axis_name="s",
                                  num_cores=NC, num_subcores=NS)


def _gather_diff(x, i0, i1):
    e = i0.shape[0]
    n, d = x.shape
    nchunk = e // CH
    nk = nchunk // NW
    ntail = nchunk - nk * NW
    assert nk % 2 == 0

    @functools.partial(
        pl.kernel,
        out_type=jax.ShapeDtypeStruct((e, d), x.dtype),
        mesh=_sc_mesh(),
        scratch_types=[
            [pltpu.VMEM((CH,), jnp.int32)] * 2,
            [pltpu.VMEM((CH,), jnp.int32)] * 2,
            [pltpu.VMEM((CH, d), x.dtype)] * 2,
            [pltpu.VMEM((CH, d), x.dtype)] * 2,
            [pltpu.SemaphoreType.DMA] * 2,
            [pltpu.SemaphoreType.DMA] * 2,
        ],
    )
    def kern(x_hbm, i0_hbm, i1_hbm, out_hbm, i0_v, i1_v, r0_v, r1_v, s0, s1):
        cid = lax.axis_index("c")
        sid = lax.axis_index("s")
        wid = sid * NC + cid

        def start(k, b):
            base = (wid + k * NW) * CH
            pltpu.sync_copy(i0_hbm.at[pl.ds(base, CH)], i0_v[b])
            pltpu.sync_copy(i1_hbm.at[pl.ds(base, CH)], i1_v[b])
            pltpu.async_copy(x_hbm.at[i1_v[b]], r1_v[b], s1[b])
            pltpu.async_copy(x_hbm.at[i0_v[b]], r0_v[b], s0[b])

        def process(k, b):
            base = (wid + k * NW) * CH
            pltpu.make_async_copy(x_hbm.at[i1_v[b]], r1_v[b], s1[b]).wait()
            pltpu.make_async_copy(x_hbm.at[i0_v[b]], r0_v[b], s0[b]).wait()

            @pl.loop(0, CH)
            def _(r):
                @pl.loop(0, d, step=16)
                def _(kk):
                    sl = (r, pl.ds(kk, 16))
                    r1_v[b][sl] = r1_v[b][sl] - r0_v[b][sl]

            pltpu.sync_copy(r1_v[b], out_hbm.at[pl.ds(base, CH)])

        start(0, 0)

        @pl.loop(0, nk, step=2)
        def _(k):
            for b in range(2):
                kk = k + b

                @pl.when(kk + 1 < nk)
                def _():
                    start(kk + 1, 1 - b)

                process(kk, b)

        @pl.when(wid < ntail)
        def _():
            start(nk, 0)
            process(nk, 0)

    return kern(x, i0, i1)


def _degree_count(i0, i1, n):
    e = i0.shape[0]
    nchunk = e // CH
    nk = nchunk // NW
    ntail = nchunk - nk * NW

    @functools.partial(
        pl.kernel,
        out_type=jax.ShapeDtypeStruct((NC, n, 128), jnp.float32),
        mesh=_sc_mesh(),
        scratch_types=[
            pltpu.VMEM((CH,), jnp.int32),
            pltpu.VMEM((CH,), jnp.int32),
            pltpu.VMEM((CH, 128), jnp.float32),
            pltpu.VMEM((CZ2, 128), jnp.float32),
            pltpu.VMEM_SHARED((n, 128), jnp.float32),
        ],
    )
    def kern(i0_hbm, i1_hbm, cnt_hbm, i0_v, i1_v, ones_v, zrow_v, acc_sh):
        cid = lax.axis_index("c")
        sid = lax.axis_index("s")
        wid = sid * NC + cid

        one = jnp.full((16,), 1.0, jnp.float32)
        zero = jnp.zeros((16,), jnp.float32)

        @pl.loop(0, CH)
        def _(r):
            @pl.loop(0, 128, step=16)
            def _(kk):
                ones_v[r, pl.ds(kk, 16)] = one

        @pl.loop(0, CZ2)
        def _(r):
            @pl.loop(0, 128, step=16)
            def _(kk):
                zrow_v[r, pl.ds(kk, 16)] = zero

        @pl.when(sid < ND)
        def _():
            for c in range(DR // CZ2):
                pltpu.sync_copy(zrow_v, acc_sh.at[pl.ds(sid * DR + c * CZ2, CZ2)])

        plsc.subcore_barrier()

        def do(k):
            base = (wid + k * NW) * CH
            pltpu.sync_copy(i0_hbm.at[pl.ds(base, CH)], i0_v)
            pltpu.sync_copy(i1_hbm.at[pl.ds(base, CH)], i1_v)
            pltpu.sync_copy(ones_v, acc_sh.at[i1_v], add=True)
            pltpu.sync_copy(ones_v, acc_sh.at[i0_v], add=True)

        @pl.loop(0, nk)
        def _(k):
            do(k)

        @pl.when(wid < ntail)
        def _():
            do(nk)

        plsc.subcore_barrier()

        @pl.when(sid < ND)
        def _():
            for c in range(DR // CZ2):
                r0 = sid * DR + c * CZ2
                pltpu.sync_copy(acc_sh.at[pl.ds(r0, CZ2)],
                                cnt_hbm.at[cid, pl.ds(r0, CZ2)])

    return kern(i0, i1)


def _message_mlp(diff, ea, w1d, w1e, b1, w2, b2, bm):
    e, d = diff.shape
    grid = (e // bm,)
    de = ea.shape[1]
    h = w1d.shape[1]
    dm = w2.shape[1]

    def body(d_ref, ea_ref, w1d_ref, w1e_ref, b1_ref, w2_ref, b2_ref, o_ref):
        q = (jnp.dot(d_ref[...], w1d_ref[...], preferred_element_type=jnp.float32)
             + jnp.dot(ea_ref[...], w1e_ref[...], preferred_element_type=jnp.float32))
        hp = jnp.maximum(q + b1_ref[...], 0.0)
        hm = jnp.maximum(b1_ref[...] - q, 0.0)
        mp = jnp.dot(hp, w2_ref[...], preferred_element_type=jnp.float32) + b2_ref[...]
        mm = jnp.dot(hm, w2_ref[...], preferred_element_type=jnp.float32) + b2_ref[...]
        o_ref[0] = jnp.maximum(mp, 0.0)
        o_ref[1] = jnp.maximum(mm, 0.0)

    return pl.pallas_call(
        body,
        grid=grid,
        in_specs=[
            pl.BlockSpec((bm, d), lambda i: (i, 0)),
            pl.BlockSpec((bm, de), lambda i: (i, 0)),
            pl.BlockSpec((d, h), lambda i: (0, 0)),
            pl.BlockSpec((de, h), lambda i: (0, 0)),
            pl.BlockSpec((1, h), lambda i: (0, 0)),
            pl.BlockSpec((h, dm), lambda i: (0, 0)),
            pl.BlockSpec((1, dm), lambda i: (0, 0)),
        ],
        out_specs=pl.BlockSpec((2, bm, dm), lambda i: (0, i, 0)),
        out_shape=jax.ShapeDtypeStruct((2, e, dm), jnp.float32),
    )(diff, ea, w1d, w1e, b1, w2, b2)


def _scatter_agg(m, i1, n):
    e2, dm = m.shape
    nchunk = e2 // CH
    nk = nchunk // NW
    ntail = nchunk - nk * NW
    assert nk % 2 == 0

    @functools.partial(
        pl.kernel,
        out_type=jax.ShapeDtypeStruct((NC, n, dm), jnp.float32),
        mesh=_sc_mesh(),
        scratch_types=[
            [pltpu.VMEM((CH,), jnp.int32)] * 2,
            [pltpu.VMEM((CH, dm), jnp.float32)] * 2,
            pltpu.VMEM((CZ2, dm), jnp.float32),
            pltpu.VMEM_SHARED((n, dm), jnp.float32),
            [pltpu.SemaphoreType.DMA] * 2,
        ],
    )
    def kern(m_hbm, i1_hbm, agg_hbm, idx_v, rows_v, zrow_v, acc_sh, sem):
        cid = lax.axis_index("c")
        sid = lax.axis_index("s")
        wid = sid * NC + cid

        zero = jnp.zeros((16,), jnp.float32)

        @pl.loop(0, CZ2)
        def _(r):
            @pl.loop(0, dm, step=16)
            def _(kk):
                zrow_v[r, pl.ds(kk, 16)] = zero

        @pl.when(sid < ND)
        def _():
            for c in range(DR // CZ2):
                pltpu.sync_copy(zrow_v, acc_sh.at[pl.ds(sid * DR + c * CZ2, CZ2)])

        plsc.subcore_barrier()

        def start(k, b):
            base = (wid + k * NW) * CH
            pltpu.sync_copy(i1_hbm.at[pl.ds(base, CH)], idx_v[b])
            pltpu.async_copy(m_hbm.at[pl.ds(base, CH)], rows_v[b], sem[b])

        def process(k, b):
            base = (wid + k * NW) * CH
            pltpu.make_async_copy(m_hbm.at[pl.ds(base, CH)], rows_v[b], sem[b]).wait()
            pltpu.sync_copy(rows_v[b], acc_sh.at[idx_v[b]], add=True)

        start(0, 0)

        @pl.loop(0, nk, step=2)
        def _(k):
            for b in range(2):
                kk = k + b

                @pl.when(kk + 1 < nk)
                def _():
                    start(kk + 1, 1 - b)

                process(kk, b)

        @pl.when(wid < ntail)
        def _():
            start(nk, 0)
            process(nk, 0)

        plsc.subcore_barrier()

        @pl.when(sid < ND)
        def _():
            for c in range(DR // CZ2):
                r0 = sid * DR + c * CZ2
                pltpu.sync_copy(acc_sh.at[pl.ds(r0, CZ2)],
                                agg_hbm.at[cid, pl.ds(r0, CZ2)])

    return kern(m, i1)


def _update_mlp(x, aggp, cntp, wu1a, wu1b, b1, wu2, b2, bn):
    n, d = x.shape
    dm = aggp.shape[2]
    h = wu1a.shape[1]
    dout = wu2.shape[1]
    grid = (n // bn,)

    def body(x_ref, a_ref, c_ref, w1a_ref, w1b_ref, b1_ref, w2_ref, b2_ref, o_ref):
        agg = a_ref[0] + a_ref[1]
        cnt = c_ref[0] + c_ref[1]
        aggm = agg / jnp.maximum(cnt, 1.0)
        g = (jnp.dot(x_ref[...], w1a_ref[...], preferred_element_type=jnp.float32)
             + jnp.dot(aggm, w1b_ref[...], preferred_element_type=jnp.float32)
             + b1_ref[...])
        u = jnp.maximum(g, 0.0)
        o_ref[...] = jnp.dot(u, w2_ref[...], preferred_element_type=jnp.float32) + b2_ref[...]

    return pl.pallas_call(
        body,
        grid=grid,
        in_specs=[
            pl.BlockSpec((bn, d), lambda i: (i, 0)),
            pl.BlockSpec((NC, bn, dm), lambda i: (0, i, 0)),
            pl.BlockSpec((NC, bn, 1), lambda i: (0, i, 0)),
            pl.BlockSpec((d, h), lambda i: (0, 0)),
            pl.BlockSpec((dm, h), lambda i: (0, 0)),
            pl.BlockSpec((1, h), lambda i: (0, 0)),
            pl.BlockSpec((h, dout), lambda i: (0, 0)),
            pl.BlockSpec((1, dout), lambda i: (0, 0)),
        ],
        out_specs=pl.BlockSpec((bn, dout), lambda i: (i, 0)),
        out_shape=jax.ShapeDtypeStruct((n, dout), jnp.float32),
    )(x, aggp, cntp, wu1a, wu1b, b1, wu2, b2)


def kernel(x, edge_attr, edge_index, W_m1, b_m1, W_m2, b_m2, W_u1, b_u1, W_u2, b_u2):
    n, d = x.shape
    e = edge_attr.shape[0]

    diff = _gather_diff(x, edge_index[0], edge_index[1])

    cntp = _degree_count(edge_index[0], edge_index[1], n)

    m2 = _message_mlp(diff, edge_attr,
                      W_m1[:d], W_m1[d:], b_m1.reshape(1, -1),
                      W_m2, b_m2.reshape(1, -1), bm=2000)
    m = m2.reshape(2 * e, -1)

    i1 = jnp.concatenate([edge_index[1], edge_index[0]], axis=0)
    aggp = _scatter_agg(m, i1, n)

    cnt = cntp[:, :, :1]

    out = _update_mlp(x, aggp, cnt,
                      W_u1[:d], W_u1[d:], b_u1.reshape(1, -1),
                      W_u2, b_u2.reshape(1, -1), bn=1000)
    return (out, None, None)


# speedup vs baseline: 9.5937x; 1.0651x over previous
import functools

import jax
import jax.numpy as jnp
from jax import lax
from jax.experimental import pallas as pl
from jax.experimental.pallas import tpu as pltpu
from jax.experimental.pallas import tpu_sc as plsc

NC = 2
NS = 16
NW = NC * NS
CH = 128
ND = 10
DR = 1000
CZ = 200
PK = 128
CZ2 = 40


def _sc_mesh():
    return plsc.VectorSubcoreMesh(core_axis_name="c", subcore_axis_name="s",
                                  num_cores=NC, num_subcores=NS)


def _gather_diff(x, i0, i1):
    e = i0.shape[0]
    n, d = x.shape
    nchunk = e // CH
    nk = nchunk // NW
    ntail = nchunk - nk * NW
    assert nk % 2 == 0

    @functools.partial(
        pl.kernel,
        out_type=jax.ShapeDtypeStruct((e, d), x.dtype),
        mesh=_sc_mesh(),
        scratch_types=[
            [pltpu.VMEM((CH,), jnp.int32)] * 2,
            [pltpu.VMEM((CH,), jnp.int32)] * 2,
            [pltpu.VMEM((CH, d), x.dtype)] * 2,
            [pltpu.VMEM((CH, d), x.dtype)] * 2,
            [pltpu.SemaphoreType.DMA] * 2,
            [pltpu.SemaphoreType.DMA] * 2,
        ],
    )
    def kern(x_hbm, i0_hbm, i1_hbm, out_hbm, i0_v, i1_v, r0_v, r1_v, s0, s1):
        cid = lax.axis_index("c")
        sid = lax.axis_index("s")
        wid = sid * NC + cid

        def start(k, b):
            base = (wid + k * NW) * CH
            pltpu.sync_copy(i0_hbm.at[pl.ds(base, CH)], i0_v[b])
            pltpu.sync_copy(i1_hbm.at[pl.ds(base, CH)], i1_v[b])
            pltpu.async_copy(x_hbm.at[i1_v[b]], r1_v[b], s1[b])
            pltpu.async_copy(x_hbm.at[i0_v[b]], r0_v[b], s0[b])

        def process(k, b):
            base = (wid + k * NW) * CH
            pltpu.make_async_copy(x_hbm.at[i1_v[b]], r1_v[b], s1[b]).wait()
            pltpu.make_async_copy(x_hbm.at[i0_v[b]], r0_v[b], s0[b]).wait()

            @pl.loop(0, CH)
            def _(r):
                @pl.loop(0, d, step=16)
                def _(kk):
                    sl = (r, pl.ds(kk, 16))
                    r1_v[b][sl] = r1_v[b][sl] - r0_v[b][sl]

            pltpu.sync_copy(r1_v[b], out_hbm.at[pl.ds(base, CH)])

        start(0, 0)

        @pl.loop(0, nk, step=2)
        def _(k):
            for b in range(2):
                kk = k + b

                @pl.when(kk + 1 < nk)
                def _():
                    start(kk + 1, 1 - b)

                process(kk, b)

        @pl.when(wid < ntail)
        def _():
            start(nk, 0)
            process(nk, 0)

    return kern(x, i0, i1)


def _degree_count(i0, i1, n):
    e = i0.shape[0]
    nchunk = e // CH
    nk = nchunk // NW
    ntail = nchunk - nk * NW

    @functools.partial(
        pl.kernel,
        out_type=jax.ShapeDtypeStruct((NC, n, 128), jnp.float32),
        mesh=_sc_mesh(),
        scratch_types=[
            pltpu.VMEM((CH,), jnp.int32),
            pltpu.VMEM((CH,), jnp.int32),
            pltpu.VMEM((CH, 128), jnp.float32),
            pltpu.VMEM((CZ2, 128), jnp.float32),
            pltpu.VMEM_SHARED((n, 128), jnp.float32),
        ],
    )
    def kern(i0_hbm, i1_hbm, cnt_hbm, i0_v, i1_v, ones_v, zrow_v, acc_sh):
        cid = lax.axis_index("c")
        sid = lax.axis_index("s")
        wid = sid * NC + cid

        one = jnp.full((16,), 1.0, jnp.float32)
        zero = jnp.zeros((16,), jnp.float32)

        @pl.loop(0, CH)
        def _(r):
            @pl.loop(0, 128, step=16)
            def _(kk):
                ones_v[r, pl.ds(kk, 16)] = one

        @pl.loop(0, CZ2)
        def _(r):
            @pl.loop(0, 128, step=16)
            def _(kk):
                zrow_v[r, pl.ds(kk, 16)] = zero

        @pl.when(sid < ND)
        def _():
            for c in range(DR // CZ2):
                pltpu.sync_copy(zrow_v, acc_sh.at[pl.ds(sid * DR + c * CZ2, CZ2)])

        plsc.subcore_barrier()

        def do(k):
            base = (wid + k * NW) * CH
            pltpu.sync_copy(i0_hbm.at[pl.ds(base, CH)], i0_v)
            pltpu.sync_copy(i1_hbm.at[pl.ds(base, CH)], i1_v)
            pltpu.sync_copy(ones_v, acc_sh.at[i1_v], add=True)
            pltpu.sync_copy(ones_v, acc_sh.at[i0_v], add=True)

        @pl.loop(0, nk)
        def _(k):
            do(k)

        @pl.when(wid < ntail)
        def _():
            do(nk)

        plsc.subcore_barrier()

        @pl.when(sid < ND)
        def _():
            for c in range(DR // CZ2):
                r0 = sid * DR + c * CZ2
                pltpu.sync_copy(acc_sh.at[pl.ds(r0, CZ2)],
                                cnt_hbm.at[cid, pl.ds(r0, CZ2)])

    return kern(i0, i1)


def _message_mlp(diff, ea, w1d, w1e, b1, w2, b2, bm):
    e, d = diff.shape
    grid = (e // bm,)
    de = ea.shape[1]
    h = w1d.shape[1]
    dm = w2.shape[1]

    def body(d_ref, ea_ref, w1d_ref, w1e_ref, b1_ref, w2_ref, b2_ref, o_ref):
        q = (jnp.dot(d_ref[...], w1d_ref[...], preferred_element_type=jnp.float32)
             + jnp.dot(ea_ref[...], w1e_ref[...], preferred_element_type=jnp.float32))
        hp = jnp.maximum(q + b1_ref[...], 0.0)
        hm = jnp.maximum(b1_ref[...] - q, 0.0)
        mp = jnp.dot(hp, w2_ref[...], preferred_element_type=jnp.float32) + b2_ref[...]
        mm = jnp.dot(hm, w2_ref[...], preferred_element_type=jnp.float32) + b2_ref[...]
        o_ref[0] = jnp.maximum(mp, 0.0)
        o_ref[1] = jnp.maximum(mm, 0.0)

    return pl.pallas_call(
        body,
        grid=grid,
        in_specs=[
            pl.BlockSpec((bm, d), lambda i: (i, 0)),
            pl.BlockSpec((bm, de), lambda i: (i, 0)),
            pl.BlockSpec((d, h), lambda i: (0, 0)),
            pl.BlockSpec((de, h), lambda i: (0, 0)),
            pl.BlockSpec((1, h), lambda i: (0, 0)),
            pl.BlockSpec((h, dm), lambda i: (0, 0)),
            pl.BlockSpec((1, dm), lambda i: (0, 0)),
        ],
        out_specs=pl.BlockSpec((2, bm, dm), lambda i: (0, i, 0)),
        out_shape=jax.ShapeDtypeStruct((2, e, dm), jnp.float32),
    )(diff, ea, w1d, w1e, b1, w2, b2)


def _scatter_agg(m, i1, n):
    e2, dm = m.shape
    nchunk = e2 // CH
    nk = nchunk // NW
    ntail = nchunk - nk * NW
    assert nk % 2 == 0

    @functools.partial(
        pl.kernel,
        out_type=jax.ShapeDtypeStruct((NC, n, dm), jnp.float32),
        mesh=_sc_mesh(),
        scratch_types=[
            [pltpu.VMEM((CH,), jnp.int32)] * 2,
            [pltpu.VMEM((CH, dm), jnp.float32)] * 2,
            pltpu.VMEM((CZ2, dm), jnp.float32),
            pltpu.VMEM_SHARED((n, dm), jnp.float32),
            [pltpu.SemaphoreType.DMA] * 2,
        ],
    )
    def kern(m_hbm, i1_hbm, agg_hbm, idx_v, rows_v, zrow_v, acc_sh, sem):
        cid = lax.axis_index("c")
        sid = lax.axis_index("s")
        wid = sid * NC + cid

        zero = jnp.zeros((16,), jnp.float32)

        @pl.loop(0, CZ2)
        def _(r):
            @pl.loop(0, dm, step=16)
            def _(kk):
                zrow_v[r, pl.ds(kk, 16)] = zero

        @pl.when(sid < ND)
        def _():
            for c in range(DR // CZ2):
                pltpu.sync_copy(zrow_v, acc_sh.at[pl.ds(sid * DR + c * CZ2, CZ2)])

        plsc.subcore_barrier()

        def start(k, b):
            base = (wid + k * NW) * CH
            pltpu.sync_copy(i1_hbm.at[pl.ds(base, CH)], idx_v[b])
            pltpu.async_copy(m_hbm.at[pl.ds(base, CH)], rows_v[b], sem[b])

        def process(k, b):
            base = (wid + k * NW) * CH
            pltpu.make_async_copy(m_hbm.at[pl.ds(base, CH)], rows_v[b], sem[b]).wait()
            pltpu.sync_copy(rows_v[b], acc_sh.at[idx_v[b]], add=True)

        start(0, 0)

        @pl.loop(0, nk, step=2)
        def _(k):
            for b in range(2):
                kk = k + b

                @pl.when(kk + 1 < nk)
                def _():
                    start(kk + 1, 1 - b)

                process(kk, b)

        @pl.when(wid < ntail)
        def _():
            start(nk, 0)
            process(nk, 0)

        plsc.subcore_barrier()

        @pl.when(sid < ND)
        def _():
            for c in range(DR // CZ2):
                r0 = sid * DR + c * CZ2
                pltpu.sync_copy(acc_sh.at[pl.ds(r0, CZ2)],
                                agg_hbm.at[cid, pl.ds(r0, CZ2)])

    return kern(m, i1)


def _update_mlp(x, aggp, aggq, cntp, wu1a, wu1b, b1, wu2, b2, bn):
    n, d = x.shape
    dm = aggp.shape[2]
    h = wu1a.shape[1]
    dout = wu2.shape[1]
    grid = (n // bn,)

    def body(x_ref, a_ref, a2_ref, c_ref, w1a_ref, w1b_ref, b1_ref, w2_ref,
             b2_ref, o_ref):
        agg = (a_ref[0] + a_ref[1]) + (a2_ref[0] + a2_ref[1])
        cnt = c_ref[0] + c_ref[1]
        aggm = agg / jnp.maximum(cnt, 1.0)
        g = (jnp.dot(x_ref[...], w1a_ref[...], preferred_element_type=jnp.float32)
             + jnp.dot(aggm, w1b_ref[...], preferred_element_type=jnp.float32)
             + b1_ref[...])
        u = jnp.maximum(g, 0.0)
        o_ref[...] = jnp.dot(u, w2_ref[...], preferred_element_type=jnp.float32) + b2_ref[...]

    return pl.pallas_call(
        body,
        grid=grid,
        in_specs=[
            pl.BlockSpec((bn, d), lambda i: (i, 0)),
            pl.BlockSpec((NC, bn, dm), lambda i: (0, i, 0)),
            pl.BlockSpec((NC, bn, dm), lambda i: (0, i, 0)),
            pl.BlockSpec((NC, bn, 1), lambda i: (0, i, 0)),
            pl.BlockSpec((d, h), lambda i: (0, 0)),
            pl.BlockSpec((dm, h), lambda i: (0, 0)),
            pl.BlockSpec((1, h), lambda i: (0, 0)),
            pl.BlockSpec((h, dout), lambda i: (0, 0)),
            pl.BlockSpec((1, dout), lambda i: (0, 0)),
        ],
        out_specs=pl.BlockSpec((bn, dout), lambda i: (i, 0)),
        out_shape=jax.ShapeDtypeStruct((n, dout), jnp.float32),
    )(x, aggp, aggq, cntp, wu1a, wu1b, b1, wu2, b2)


def kernel(x, edge_attr, edge_index, W_m1, b_m1, W_m2, b_m2, W_u1, b_u1, W_u2, b_u2):
    n, d = x.shape
    e = edge_attr.shape[0]

    ea_half = 163840
    slabs = ((0, ea_half), (ea_half, e))

    mlp = functools.partial(_message_mlp,
                            w1d=W_m1[:d], w1e=W_m1[d:], b1=b_m1.reshape(1, -1),
                            w2=W_m2, b2=b_m2.reshape(1, -1), bm=1280)

    diffs = [_gather_diff(x, edge_index[0, lo:hi], edge_index[1, lo:hi])
             for lo, hi in slabs]

    cntp = _degree_count(edge_index[0], edge_index[1], n)

    m2s = [mlp(diffs[j], edge_attr[lo:hi]) for j, (lo, hi) in enumerate(slabs)]

    aggps = []
    for j, (lo, hi) in enumerate(slabs):
        i1j = jnp.concatenate([edge_index[1, lo:hi], edge_index[0, lo:hi]])
        aggps.append(_scatter_agg(m2s[j].reshape(2 * (hi - lo), -1), i1j, n))

    cnt = cntp[:, :, :1]

    out = _update_mlp(x, aggps[0], aggps[1], cnt,
                      W_u1[:d], W_u1[d:], b_u1.reshape(1, -1),
                      W_u2, b_u2.reshape(1, -1), bn=1000)
    return (out, None, None)
